# Initial kernel scaffold; baseline (speedup 1.0000x reference)
import functools

import jax
import jax.numpy as jnp
from jax import lax
from jax.experimental import pallas as pl
from jax.experimental.pallas import tpu as pltpu
from jax.experimental.pallas import tpu_sc as plsc

_NC = 2
_NS = 16
_NW = _NC * _NS
_L = 16
_ROUNDS = 6


def _gather_dedup(mem, src2d, dst2d, n_rows, b):
    d = mem.shape[1]
    kpw = b // _NW // 128
    kps = b // _NS // 128
    mesh = plsc.VectorSubcoreMesh(core_axis_name="c", subcore_axis_name="s")

    @functools.partial(
        pl.kernel,
        out_type=[
            jax.ShapeDtypeStruct((b, d), jnp.float32),
            jax.ShapeDtypeStruct((b, d), jnp.float32),
            jax.ShapeDtypeStruct((b // 128, 128), jnp.int32),
        ],
        mesh=mesh,
        scratch_types=[
            pltpu.VMEM((kpw, 128), jnp.int32),
            pltpu.VMEM((kpw, 128), jnp.int32),
            pltpu.VMEM((kpw, 128, d), jnp.float32),
            pltpu.VMEM((kpw, 128, d), jnp.float32),
            pltpu.VMEM((kps, 128), jnp.int32),
            pltpu.VMEM((kps, 128), jnp.int32),
            pltpu.VMEM((kps, 128), jnp.int32),
            pltpu.VMEM((kps, 128), jnp.int32),
            pltpu.VMEM_SHARED((n_rows + _L,), jnp.int32),
            pltpu.SemaphoreType.DMA,
        ],
    )
    def body(src_hbm, dst_hbm, mem_hbm, smem_out, dmem_out, win_out,
             sidx, didx, srows, drows, dsub, pos, tcur, swin, tag, sem):
        cid = lax.axis_index("c")
        sid = lax.axis_index("s")
        wid = sid * _NC + cid
        gbase = wid * kpw

        pltpu.sync_copy(src_hbm.at[pl.ds(gbase, kpw)], sidx)
        pltpu.sync_copy(dst_hbm.at[pl.ds(gbase, kpw)], didx)
        cps = []
        for j in range(kpw):
            cps.append(pltpu.async_copy(mem_hbm.at[sidx.at[j]], srows.at[j], sem))
            cps.append(pltpu.async_copy(mem_hbm.at[didx.at[j]], drows.at[j], sem))
        for c in cps:
            c.wait()
        cps = []
        for j in range(kpw):
            r0 = (gbase + j) * 128
            cps.append(pltpu.async_copy(srows.at[j], smem_out.at[pl.ds(r0, 128)], sem))
            cps.append(pltpu.async_copy(drows.at[j], dmem_out.at[pl.ds(r0, 128)], sem))
        for c in cps:
            c.wait()

        dbase = sid * kps
        pltpu.sync_copy(dst_hbm.at[pl.ds(dbase, kps)], dsub)
        for j in range(kps):
            for k in range(128 // _L):
                pos[j, pl.ds(k * _L, _L)] = (
                    lax.iota(jnp.int32, (_L,)) + ((dbase + j) * 128 + k * _L)
                )
        cps = [pltpu.async_copy(pos.at[j], tag.at[dsub.at[j]], sem)
               for j in range(kps)]
        for c in cps:
            c.wait()
        plsc.subcore_barrier()
        for _ in range(_ROUNDS):
            cps = [pltpu.async_copy(tag.at[dsub.at[j]], tcur.at[j], sem)
                   for j in range(kps)]
            for c in cps:
                c.wait()
            plsc.subcore_barrier()
            for j in range(kps):
                for k in range(128 // _L):
                    sl = pl.ds(k * _L, _L)
                    p = pos[j, sl]
                    t = tcur[j, sl]
                    dump = n_rows + lax.iota(jnp.int32, (_L,))
                    swin[j, sl] = jnp.where(p > t, dsub[j, sl], dump)
            cps = [pltpu.async_copy(pos.at[j], tag.at[swin.at[j]], sem)
                   for j in range(kps)]
            for c in cps:
                c.wait()
            plsc.subcore_barrier()
        cps = [pltpu.async_copy(tag.at[dsub.at[j]], tcur.at[j], sem)
               for j in range(kps)]
        for c in cps:
            c.wait()
        pltpu.sync_copy(tcur, win_out.at[pl.ds(dbase, kps)])

    return body(src2d, dst2d, mem)


def _dense_body(s_ref, dm_ref, e_ref, w1s_ref, w1d_ref, w1e_ref, b1_ref,
                w2_ref, b2_ref, wih_ref, whh_ref, bih_ref, bhh_ref, o_ref):
    f32 = jnp.float32
    hp = lax.Precision.HIGHEST
    s = s_ref[...]
    dm = dm_ref[...]
    e = e_ref[...]
    d = o_ref.shape[1]
    h = (jnp.dot(s, w1s_ref[...], precision=hp, preferred_element_type=f32)
         + jnp.dot(dm, w1d_ref[...], precision=hp, preferred_element_type=f32)
         + jnp.dot(e, w1e_ref[...], precision=hp, preferred_element_type=f32)
         + b1_ref[...])
    h = jnp.maximum(h, 0.0)
    msg = jnp.dot(h, w2_ref[...], precision=hp, preferred_element_type=f32) + b2_ref[...]
    a = jnp.dot(msg, wih_ref[...], precision=hp, preferred_element_type=f32) + bih_ref[...]
    g = jnp.dot(dm, whh_ref[...], precision=hp, preferred_element_type=f32) + bhh_ref[...]
    r = jax.nn.sigmoid(a[:, :d] + g[:, :d])
    z = jax.nn.sigmoid(a[:, d:2 * d] + g[:, d:2 * d])
    n = jnp.tanh(a[:, 2 * d:] + r * g[:, 2 * d:])
    o_ref[...] = (1.0 - z) * n + z * dm


def _dense(smem, dmem, edge_feats, W1, b1, W2, b2, Wih, Whh, bih, bhh):
    b, d = smem.shape
    ef = edge_feats.shape[1]
    w1t = W1.T
    w1s, w1d, w1e = w1t[:d], w1t[d:2 * d], w1t[2 * d:]
    blk = 2048
    rep = lambda i: (0, 0)
    row = lambda i: (i, 0)
    return pl.pallas_call(
        _dense_body,
        grid=(b // blk,),
        in_specs=[
            pl.BlockSpec((blk, d), row),
            pl.BlockSpec((blk, d), row),
            pl.BlockSpec((blk, ef), row),
            pl.BlockSpec((d, d), rep),
            pl.BlockSpec((d, d), rep),
            pl.BlockSpec((ef, d), rep),
            pl.BlockSpec((1, d), rep),
            pl.BlockSpec((d, d), rep),
            pl.BlockSpec((1, d), rep),
            pl.BlockSpec((d, 3 * d), rep),
            pl.BlockSpec((d, 3 * d), rep),
            pl.BlockSpec((1, 3 * d), rep),
            pl.BlockSpec((1, 3 * d), rep),
        ],
        out_specs=pl.BlockSpec((blk, d), row),
        out_shape=jax.ShapeDtypeStruct((b, d), jnp.float32),
    )(smem, dmem, edge_feats, w1s, w1d, w1e, b1.reshape(1, d),
      W2.T, b2.reshape(1, d), Wih.T, Whh.T,
      bih.reshape(1, 3 * d), bhh.reshape(1, 3 * d))


def _scatter(mem_ref, upd, win2d, dst2d, b):
    d = upd.shape[1]
    kpw = b // _NW // 128
    mesh = plsc.VectorSubcoreMesh(core_axis_name="c", subcore_axis_name="s")

    @functools.partial(
        pl.kernel,
        out_type=(),
        mesh=mesh,
        scratch_types=[
            pltpu.VMEM((kpw, 128), jnp.int32),
            pltpu.VMEM((kpw, 128), jnp.int32),
            pltpu.VMEM((kpw, 128, d), jnp.float32),
            pltpu.SemaphoreType.DMA,
        ],
    )
    def body(upd_hbm, win_hbm, dst_hbm, mem_hbm, wv, dv, rows, sem):
        cid = lax.axis_index("c")
        sid = lax.axis_index("s")
        wid = sid * _NC + cid
        gbase = wid * kpw
        pltpu.sync_copy(win_hbm.at[pl.ds(gbase, kpw)], wv)
        pltpu.sync_copy(dst_hbm.at[pl.ds(gbase, kpw)], dv)
        cps = [pltpu.async_copy(upd_hbm.at[wv.at[j]], rows.at[j], sem)
               for j in range(kpw)]
        for c in cps:
            c.wait()
        cps = [pltpu.async_copy(rows.at[j], mem_hbm.at[dv.at[j]], sem)
               for j in range(kpw)]
        for c in cps:
            c.wait()

    body(upd, win2d, dst2d, mem_ref)


def kernel(src_ids, dst_ids, edge_feats, timestamps, memory, W1, b1, W2, b2,
           Wih, Whh, bih, bhh):
    n_rows, d = memory.shape
    b = src_ids.shape[0]
    src2d = src_ids.astype(jnp.int32).reshape(b // 128, 128)
    dst2d = dst_ids.astype(jnp.int32).reshape(b // 128, 128)
    smem, dmem, win2d = _gather_dedup(memory, src2d, dst2d, n_rows, b)
    upd = _dense(smem, dmem, edge_feats, W1, b1, W2, b2, Wih, Whh, bih, bhh)
    mem_ref = jax.new_ref(memory)
    _scatter(mem_ref, upd, win2d, dst2d, b)
    return mem_ref[...]


# speedup vs baseline: 2.0340x; 2.0340x over previous
import functools

import jax
import jax.numpy as jnp
from jax import lax
from jax.experimental import pallas as pl
from jax.experimental.pallas import tpu as pltpu
from jax.experimental.pallas import tpu_sc as plsc

_NC = 2
_NS = 16
_NW = _NC * _NS
_L = 16
_ROUNDS = 6


def _gather_dedup(mem, src2d, dst2d, n_rows, b):
    d = mem.shape[1]
    kpw = b // _NW // 128
    kps = b // _NS // 128
    mesh = plsc.VectorSubcoreMesh(core_axis_name="c", subcore_axis_name="s")

    @functools.partial(
        pl.kernel,
        out_type=[
            jax.ShapeDtypeStruct((b, d), jnp.float32),
            jax.ShapeDtypeStruct((b, d), jnp.float32),
            jax.ShapeDtypeStruct((b // 128, 128), jnp.int32),
        ],
        mesh=mesh,
        compiler_params=pltpu.CompilerParams(use_tc_tiling_on_sc=False),
        scratch_types=[
            pltpu.VMEM((kpw, 128), jnp.int32),
            pltpu.VMEM((kpw, 128), jnp.int32),
            pltpu.VMEM((kpw, 128, d), jnp.float32),
            pltpu.VMEM((kpw, 128, d), jnp.float32),
            pltpu.VMEM((kps, 128), jnp.int32),
            pltpu.VMEM((kps, 128), jnp.int32),
            pltpu.VMEM((kps, 128), jnp.int32),
            pltpu.VMEM((kps, 128), jnp.int32),
            pltpu.VMEM_SHARED((n_rows + _L,), jnp.int32),
            pltpu.SemaphoreType.DMA,
        ],
    )
    def body(src_hbm, dst_hbm, mem_hbm, smem_out, dmem_out, win_out,
             sidx, didx, srows, drows, dsub, pos, tcur, swin, tag, sem):
        cid = lax.axis_index("c")
        sid = lax.axis_index("s")
        wid = sid * _NC + cid
        gbase = wid * kpw

        pltpu.sync_copy(src_hbm.at[pl.ds(gbase, kpw)], sidx)
        pltpu.sync_copy(dst_hbm.at[pl.ds(gbase, kpw)], didx)
        cps = []
        for j in range(kpw):
            cps.append(pltpu.async_copy(mem_hbm.at[sidx.at[j]], srows.at[j], sem))
            cps.append(pltpu.async_copy(mem_hbm.at[didx.at[j]], drows.at[j], sem))
        for c in cps:
            c.wait()
        cps = []
        for j in range(kpw):
            r0 = (gbase + j) * 128
            cps.append(pltpu.async_copy(srows.at[j], smem_out.at[pl.ds(r0, 128)], sem))
            cps.append(pltpu.async_copy(drows.at[j], dmem_out.at[pl.ds(r0, 128)], sem))
        for c in cps:
            c.wait()

        dbase = sid * kps
        pltpu.sync_copy(dst_hbm.at[pl.ds(dbase, kps)], dsub)
        for j in range(kps):
            for k in range(128 // _L):
                pos[j, pl.ds(k * _L, _L)] = (
                    lax.iota(jnp.int32, _L) + ((dbase + j) * 128 + k * _L)
                )
        cps = [pltpu.async_copy(pos.at[j], tag.at[dsub.at[j]], sem)
               for j in range(kps)]
        for c in cps:
            c.wait()
        plsc.subcore_barrier()
        for _ in range(_ROUNDS):
            cps = [pltpu.async_copy(tag.at[dsub.at[j]], tcur.at[j], sem)
                   for j in range(kps)]
            for c in cps:
                c.wait()
            plsc.subcore_barrier()
            for j in range(kps):
                for k in range(128 // _L):
                    sl = pl.ds(k * _L, _L)
                    p = pos[j, sl]
                    t = tcur[j, sl]
                    dump = n_rows + lax.iota(jnp.int32, _L)
                    swin[j, sl] = jnp.where(p > t, dsub[j, sl], dump)
            cps = [pltpu.async_copy(pos.at[j], tag.at[swin.at[j]], sem)
                   for j in range(kps)]
            for c in cps:
                c.wait()
            plsc.subcore_barrier()
        cps = [pltpu.async_copy(tag.at[dsub.at[j]], tcur.at[j], sem)
               for j in range(kps)]
        for c in cps:
            c.wait()
        pltpu.sync_copy(tcur, win_out.at[pl.ds(dbase, kps)])

    return body(src2d, dst2d, mem)


def _dense_body(s_ref, dm_ref, e_ref, w1s_ref, w1d_ref, w1e_ref, b1_ref,
                w2_ref, b2_ref, wih_ref, whh_ref, bih_ref, bhh_ref, o_ref):
    f32 = jnp.float32
    hp = lax.Precision.HIGHEST
    s = s_ref[...]
    dm = dm_ref[...]
    e = e_ref[...]
    d = o_ref.shape[1]
    h = (jnp.dot(s, w1s_ref[...], precision=hp, preferred_element_type=f32)
         + jnp.dot(dm, w1d_ref[...], precision=hp, preferred_element_type=f32)
         + jnp.dot(e, w1e_ref[...], precision=hp, preferred_element_type=f32)
         + b1_ref[...])
    h = jnp.maximum(h, 0.0)
    msg = jnp.dot(h, w2_ref[...], precision=hp, preferred_element_type=f32) + b2_ref[...]
    a = jnp.dot(msg, wih_ref[...], precision=hp, preferred_element_type=f32) + bih_ref[...]
    g = jnp.dot(dm, whh_ref[...], precision=hp, preferred_element_type=f32) + bhh_ref[...]
    r = jax.nn.sigmoid(a[:, :d] + g[:, :d])
    z = jax.nn.sigmoid(a[:, d:2 * d] + g[:, d:2 * d])
    n = jnp.tanh(a[:, 2 * d:] + r * g[:, 2 * d:])
    o_ref[...] = (1.0 - z) * n + z * dm


def _dense(smem, dmem, edge_feats, W1, b1, W2, b2, Wih, Whh, bih, bhh):
    b, d = smem.shape
    ef = edge_feats.shape[1]
    w1t = W1.T
    w1s, w1d, w1e = w1t[:d], w1t[d:2 * d], w1t[2 * d:]
    blk = 2048
    rep = lambda i: (0, 0)
    row = lambda i: (i, 0)
    return pl.pallas_call(
        _dense_body,
        grid=(b // blk,),
        in_specs=[
            pl.BlockSpec((blk, d), row),
            pl.BlockSpec((blk, d), row),
            pl.BlockSpec((blk, ef), row),
            pl.BlockSpec((d, d), rep),
            pl.BlockSpec((d, d), rep),
            pl.BlockSpec((ef, d), rep),
            pl.BlockSpec((1, d), rep),
            pl.BlockSpec((d, d), rep),
            pl.BlockSpec((1, d), rep),
            pl.BlockSpec((d, 3 * d), rep),
            pl.BlockSpec((d, 3 * d), rep),
            pl.BlockSpec((1, 3 * d), rep),
            pl.BlockSpec((1, 3 * d), rep),
        ],
        out_specs=pl.BlockSpec((blk, d), row),
        out_shape=jax.ShapeDtypeStruct((b, d), jnp.float32),
    )(smem, dmem, edge_feats, w1s, w1d, w1e, b1.reshape(1, d),
      W2.T, b2.reshape(1, d), Wih.T, Whh.T,
      bih.reshape(1, 3 * d), bhh.reshape(1, 3 * d))


def _scatter(mem_ref, upd, win2d, dst2d, b):
    d = upd.shape[1]
    kpw = b // _NW // 128
    mesh = plsc.VectorSubcoreMesh(core_axis_name="c", subcore_axis_name="s")

    @functools.partial(
        pl.kernel,
        out_type=(),
        mesh=mesh,
        compiler_params=pltpu.CompilerParams(use_tc_tiling_on_sc=False),
        scratch_types=[
            pltpu.VMEM((kpw, 128), jnp.int32),
            pltpu.VMEM((kpw, 128), jnp.int32),
            pltpu.VMEM((kpw, 128, d), jnp.float32),
            pltpu.SemaphoreType.DMA,
        ],
    )
    def body(upd_hbm, win_hbm, dst_hbm, mem_hbm, wv, dv, rows, sem):
        cid = lax.axis_index("c")
        sid = lax.axis_index("s")
        wid = sid * _NC + cid
        gbase = wid * kpw
        pltpu.sync_copy(win_hbm.at[pl.ds(gbase, kpw)], wv)
        pltpu.sync_copy(dst_hbm.at[pl.ds(gbase, kpw)], dv)
        cps = [pltpu.async_copy(upd_hbm.at[wv.at[j]], rows.at[j], sem)
               for j in range(kpw)]
        for c in cps:
            c.wait()
        cps = [pltpu.async_copy(rows.at[j], mem_hbm.at[dv.at[j]], sem)
               for j in range(kpw)]
        for c in cps:
            c.wait()

    body(upd, win2d, dst2d, mem_ref)


def kernel(src_ids, dst_ids, edge_feats, timestamps, memory, W1, b1, W2, b2,
           Wih, Whh, bih, bhh):
    n_rows, d = memory.shape
    b = src_ids.shape[0]
    src2d = src_ids.astype(jnp.int32).reshape(b // 128, 128)
    dst2d = dst_ids.astype(jnp.int32).reshape(b // 128, 128)
    smem, dmem, win2d = _gather_dedup(memory, src2d, dst2d, n_rows, b)
    upd = _dense(smem, dmem, edge_feats, W1, b1, W2, b2, Wih, Whh, bih, bhh)
    mem_ref = jax.new_ref(memory)
    _scatter(mem_ref, upd, win2d, dst2d, b)
    return mem_ref[...]
